# baseline (speedup 1.0000x reference)
import functools

import jax
import jax.numpy as jnp
from jax import lax
from jax.experimental import pallas as pl
from jax.experimental.pallas import tpu as pltpu
from jax.experimental.pallas import tpu_sc as plsc

B = 16384
DD = 13
DC = 26
CARD = 100000
D = 64
NT = DC + 1

W = 2048
SH = 11
NCH = CARD // W
NBK = NCH + 1
NBP = 64
LW = B // 16


def _dense_body(wt_ref, x_ref, b_ref, o_ref):
    o_ref[...] = (
        jnp.dot(wt_ref[...], x_ref[...], preferred_element_type=jnp.float32)
        + b_ref[...]
    )


def _dense_proj_t(dense_t, WdT, bd):
    grid = 8
    blk = B // grid
    return pl.pallas_call(
        _dense_body,
        grid=(grid,),
        in_specs=[
            pl.BlockSpec((D, DD), lambda i: (0, 0)),
            pl.BlockSpec((DD, blk), lambda i: (0, i)),
            pl.BlockSpec((D, 1), lambda i: (0, 0)),
        ],
        out_specs=pl.BlockSpec((D, blk), lambda i: (0, i)),
        out_shape=jax.ShapeDtypeStruct((D, B), jnp.float32),
    )(WdT, dense_t, bd.reshape(D, 1))


def _make_sc_kernel(nc, ns):
    assert nc == 2 and ns == 16
    nu = DC * 8 // ns
    mesh = plsc.VectorSubcoreMesh(core_axis_name="c", subcore_axis_name="s")

    @functools.partial(
        pl.kernel,
        out_type=jax.ShapeDtypeStruct((NT, D, B), jnp.float32),
        mesh=mesh,
        scratch_types=[
            pltpu.VMEM((2048,), jnp.int32),
            pltpu.VMEM((B,), jnp.int32),
            pltpu.VMEM((NBP * 16,), jnp.int32),
            pltpu.VMEM((NBP,), jnp.int32),
            pltpu.VMEM((4, B), jnp.float32),
            pltpu.VMEM((2, 4, W), jnp.float32),
            pltpu.VMEM_SHARED((DC * B,), jnp.int32),
            pltpu.VMEM_SHARED((DC * NBP,), jnp.int32),
            pltpu.SemaphoreType.DMA,
            pltpu.SemaphoreType.DMA,
            pltpu.SemaphoreType.DMA,
            pltpu.SemaphoreType.DMA,
            pltpu.SemaphoreType.DMA,
            pltpu.SemaphoreType.DMA,
            pltpu.SemaphoreType.DMA,
            pltpu.SemaphoreType.DMA,
        ],
        compiler_params=pltpu.CompilerParams(needs_layout_passes=False),
    )
    def sc_kernel(sp_hbm, tab_hbm, tail_hbm, dtok_hbm, out_hbm,
                  sidx, bl_v, cnt_v, off_v, obuf, stage, sh_bl, sh_off,
                  sem0, sem1, sem2, sem3, sem4, sem5, sem6, sem7):
        cid = lax.axis_index("c")
        sid = lax.axis_index("s")
        wid = sid * nc + cid
        lane = lax.iota(jnp.int32, 16)

        @pl.when(wid < 16)
        def _():
            d0 = wid * 4
            pltpu.sync_copy(dtok_hbm.at[pl.ds(d0, 4), :], obuf)
            pltpu.sync_copy(obuf, out_hbm.at[DC, pl.ds(d0, 4), :])

        for rep in range(2):
            f = sid + rep * ns

            @pl.when(f < DC)
            def _():
                zero = jnp.zeros((16,), jnp.int32)

                @pl.loop(0, NBP)
                def _(q):
                    cnt_v[pl.ds(q * 16, 16)] = zero

                for win in range(B // 2048):
                    pltpu.sync_copy(
                        sp_hbm.at[f, pl.ds(win * 2048, 2048)], sidx)

                    @pl.loop(0, 128)
                    def _(t):
                        v = sidx[pl.ds(t * 16, 16)]
                        addr = lax.shift_right_logical(v, SH) * 16 + lane
                        cur = plsc.load_gather(cnt_v, [addr])
                        plsc.store_scatter(cnt_v, [addr], cur + 1)

                @pl.loop(0, NBP, init_carry=jnp.int32(0))
                def _(q, carry):
                    c = cnt_v[pl.ds(q * 16, 16)]
                    inc = plsc.cumsum(c)
                    cnt_v[pl.ds(q * 16, 16)] = inc - c + carry
                    return carry + jnp.sum(c)

                for q16 in range(NBP // 16):
                    a0 = (q16 * 16 + lane) * 16
                    off_v[pl.ds(q16 * 16, 16)] = plsc.load_gather(cnt_v, [a0])

                for win in range(B // 2048):
                    pltpu.sync_copy(
                        sp_hbm.at[f, pl.ds(win * 2048, 2048)], sidx)

                    @pl.loop(0, 128)
                    def _(t):
                        v = sidx[pl.ds(t * 16, 16)]
                        addr = lax.shift_right_logical(v, SH) * 16 + lane
                        p = plsc.load_gather(cnt_v, [addr])
                        b = win * 2048 + t * 16 + lane
                        val = b + lax.shift_left(
                            jnp.bitwise_and(v, W - 1), 14)
                        plsc.store_scatter(bl_v, [p], val)
                        plsc.store_scatter(cnt_v, [addr], p + 1)

                pltpu.sync_copy(bl_v, sh_bl.at[pl.ds(f * B, B)])
                pltpu.sync_copy(off_v, sh_off.at[pl.ds(f * NBP, NBP)])

        plsc.subcore_barrier()

        HW = W // 4
        bufsems = [[sem0, sem2, sem4, sem6], [sem1, sem3, sem5, sem7]]

        def start_chunk(k, buf, sems):
            for q in range(4):
                pltpu.async_copy(
                    tab_hbm.at[f2, pl.ds(d0, 4), pl.ds(k * W + q * HW, HW)],
                    stage.at[buf, :, pl.ds(q * HW, HW)], sems[q])

        def wait_chunk(k, buf, sems):
            for q in range(4):
                pltpu.make_async_copy(
                    tab_hbm.at[f2, pl.ds(d0, 4), pl.ds(k * W + q * HW, HW)],
                    stage.at[buf, :, pl.ds(q * HW, HW)], sems[q]).wait()

        def process(k, buf):
            ka = jnp.full((16,), k, jnp.int32) + jnp.minimum(lane, 1)
            vo = plsc.load_gather(off_v, [ka])
            s_lo = vo[0]
            s_hi = vo[1]
            ng = lax.div(s_hi - s_lo + 15, 16)
            sref = stage.at[buf]

            @plsc.parallel_loop(0, ng, unroll=2)
            def _(g):
                e = jnp.minimum(s_lo + g * 16 + lane, B - 1)
                m = s_lo + g * 16 + lane < s_hi
                ev = plsc.load_gather(bl_v, [e], mask=m)
                bv = jnp.bitwise_and(ev, 0x3FFF)
                jl = lax.shift_right_logical(ev, 14)
                for dl in range(4):
                    dv = jnp.full((16,), dl, jnp.int32)
                    val = plsc.load_gather(sref, [dv, jl], mask=m)
                    plsc.store_scatter(obuf, [dv, bv], val, mask=m)

        for t in range(nu):
            u = t * ns + sid
            f2 = lax.div(u, 8)
            d0 = cid * 32 + (u - f2 * 8) * 4
            pltpu.sync_copy(sh_bl.at[pl.ds(f2 * B, B)], bl_v)
            pltpu.sync_copy(sh_off.at[pl.ds(f2 * NBP, NBP)], off_v)

            start_chunk(jnp.int32(0), 0, bufsems[0])

            @pl.loop(0, NCH, step=2)
            def _(k):
                wait_chunk(k, 0, bufsems[0])
                start_chunk(k + 1, 1, bufsems[1])
                process(k, 0)
                wait_chunk(k + 1, 1, bufsems[1])

                @pl.when(k + 2 < NCH)
                def _():
                    start_chunk(k + 2, 0, bufsems[0])

                process(k + 1, 1)

            pltpu.sync_copy(tail_hbm.at[f2, pl.ds(d0, 4), :], stage.at[0])
            process(NCH, 0)
            pltpu.sync_copy(obuf, out_hbm.at[f2, pl.ds(d0, 4), :])

    return sc_kernel


def kernel(dense, sparse, cat_embs, Wd, bd):
    info = plsc.get_sparse_core_info()
    dtok_t = _dense_proj_t(dense.T, Wd.T, bd)
    sc_k = _make_sc_kernel(info.num_cores, info.num_subcores)
    tab_t = jnp.transpose(cat_embs, (0, 2, 1))
    tail = jnp.pad(tab_t[:, :, NCH * W:], ((0, 0), (0, 0), (0, W - (CARD - NCH * W))))
    out3 = sc_k(
        sparse.T,
        tab_t,
        tail,
        dtok_t,
    )
    return jnp.transpose(out3, (2, 0, 1))


# speedup vs baseline: 1.0368x; 1.0368x over previous
import functools

import jax
import jax.numpy as jnp
from jax import lax
from jax.experimental import pallas as pl
from jax.experimental.pallas import tpu as pltpu
from jax.experimental.pallas import tpu_sc as plsc

B = 16384
DD = 13
DC = 26
CARD = 100000
D = 64
NT = DC + 1

W = 2048
SH = 11
NCH = CARD // W
NBK = NCH + 1
NBP = 64
LW = B // 16


def _dense_body(wt_ref, x_ref, b_ref, o_ref):
    o_ref[...] = (
        jnp.dot(wt_ref[...], x_ref[...], preferred_element_type=jnp.float32)
        + b_ref[...]
    )


def _dense_proj_t(dense_t, WdT, bd):
    grid = 8
    blk = B // grid
    return pl.pallas_call(
        _dense_body,
        grid=(grid,),
        in_specs=[
            pl.BlockSpec((D, DD), lambda i: (0, 0)),
            pl.BlockSpec((DD, blk), lambda i: (0, i)),
            pl.BlockSpec((D, 1), lambda i: (0, 0)),
        ],
        out_specs=pl.BlockSpec((D, blk), lambda i: (0, i)),
        out_shape=jax.ShapeDtypeStruct((D, B), jnp.float32),
    )(WdT, dense_t, bd.reshape(D, 1))


def _make_sc_kernel(nc, ns):
    assert nc == 2 and ns == 16
    nu = DC * 8 // ns
    mesh = plsc.VectorSubcoreMesh(core_axis_name="c", subcore_axis_name="s")

    @functools.partial(
        pl.kernel,
        out_type=jax.ShapeDtypeStruct((NT, D, B), jnp.float32),
        mesh=mesh,
        scratch_types=[
            pltpu.VMEM((2048,), jnp.int32),
            pltpu.VMEM((B,), jnp.int32),
            pltpu.VMEM((NBP * 16,), jnp.int32),
            pltpu.VMEM((NBP,), jnp.int32),
            pltpu.VMEM((4, B), jnp.float32),
            pltpu.VMEM((2, 4, W), jnp.float32),
            pltpu.VMEM_SHARED((DC * B,), jnp.int32),
            pltpu.VMEM_SHARED((DC * NBP,), jnp.int32),
            pltpu.SemaphoreType.DMA,
            pltpu.SemaphoreType.DMA,
            pltpu.SemaphoreType.DMA,
            pltpu.SemaphoreType.DMA,
        ],
        compiler_params=pltpu.CompilerParams(needs_layout_passes=False),
    )
    def sc_kernel(sp_hbm, tab_hbm, tail_hbm, dtok_hbm, out_hbm,
                  sidx, bl_v, cnt_v, off_v, obuf, stage, sh_bl, sh_off,
                  sem0, sem1, sem2, sem3):
        cid = lax.axis_index("c")
        sid = lax.axis_index("s")
        wid = sid * nc + cid
        lane = lax.iota(jnp.int32, 16)

        @pl.when(wid < 16)
        def _():
            d0 = wid * 4
            pltpu.sync_copy(dtok_hbm.at[pl.ds(d0, 4), :], obuf)
            pltpu.sync_copy(obuf, out_hbm.at[DC, pl.ds(d0, 4), :])

        for rep in range(2):
            f = sid + rep * ns

            @pl.when(f < DC)
            def _():
                zero = jnp.zeros((16,), jnp.int32)

                @pl.loop(0, NBP)
                def _(q):
                    cnt_v[pl.ds(q * 16, 16)] = zero

                for win in range(B // 2048):
                    pltpu.sync_copy(
                        sp_hbm.at[f, pl.ds(win * 2048, 2048)], sidx)

                    @pl.loop(0, 128)
                    def _(t):
                        v = sidx[pl.ds(t * 16, 16)]
                        addr = lax.shift_right_logical(v, SH) * 16 + lane
                        cur = plsc.load_gather(cnt_v, [addr])
                        plsc.store_scatter(cnt_v, [addr], cur + 1)

                @pl.loop(0, NBP, init_carry=jnp.int32(0))
                def _(q, carry):
                    c = cnt_v[pl.ds(q * 16, 16)]
                    inc = plsc.cumsum(c)
                    cnt_v[pl.ds(q * 16, 16)] = inc - c + carry
                    return carry + jnp.sum(c)

                for q16 in range(NBP // 16):
                    a0 = (q16 * 16 + lane) * 16
                    off_v[pl.ds(q16 * 16, 16)] = plsc.load_gather(cnt_v, [a0])

                for win in range(B // 2048):
                    pltpu.sync_copy(
                        sp_hbm.at[f, pl.ds(win * 2048, 2048)], sidx)

                    @pl.loop(0, 128)
                    def _(t):
                        v = sidx[pl.ds(t * 16, 16)]
                        addr = lax.shift_right_logical(v, SH) * 16 + lane
                        p = plsc.load_gather(cnt_v, [addr])
                        b = win * 2048 + t * 16 + lane
                        val = b + lax.shift_left(
                            jnp.bitwise_and(v, W - 1), 14)
                        plsc.store_scatter(bl_v, [p], val)
                        plsc.store_scatter(cnt_v, [addr], p + 1)

                pltpu.sync_copy(bl_v, sh_bl.at[pl.ds(f * B, B)])
                pltpu.sync_copy(off_v, sh_off.at[pl.ds(f * NBP, NBP)])

        plsc.subcore_barrier()

        HW = W // 2

        def start_chunk(k, buf, sema, semb):
            pltpu.async_copy(
                tab_hbm.at[f2, pl.ds(d0, 4), pl.ds(k * W, HW)],
                stage.at[buf, :, pl.ds(0, HW)], sema)
            pltpu.async_copy(
                tab_hbm.at[f2, pl.ds(d0, 4), pl.ds(k * W + HW, HW)],
                stage.at[buf, :, pl.ds(HW, HW)], semb)

        def wait_chunk(k, buf, sema, semb):
            pltpu.make_async_copy(
                tab_hbm.at[f2, pl.ds(d0, 4), pl.ds(k * W, HW)],
                stage.at[buf, :, pl.ds(0, HW)], sema).wait()
            pltpu.make_async_copy(
                tab_hbm.at[f2, pl.ds(d0, 4), pl.ds(k * W + HW, HW)],
                stage.at[buf, :, pl.ds(HW, HW)], semb).wait()

        def process(k, buf):
            ka = jnp.full((16,), k, jnp.int32) + jnp.minimum(lane, 1)
            vo = plsc.load_gather(off_v, [ka])
            s_lo = vo[0]
            s_hi = vo[1]
            ng = lax.div(s_hi - s_lo + 15, 16)
            sref = stage.at[buf]

            @plsc.parallel_loop(0, ng, unroll=2)
            def _(g):
                e = jnp.minimum(s_lo + g * 16 + lane, B - 1)
                m = s_lo + g * 16 + lane < s_hi
                ev = plsc.load_gather(bl_v, [e], mask=m)
                bv = jnp.bitwise_and(ev, 0x3FFF)
                jl = lax.shift_right_logical(ev, 14)
                for dl in range(4):
                    dv = jnp.full((16,), dl, jnp.int32)
                    val = plsc.load_gather(sref, [dv, jl], mask=m)
                    plsc.store_scatter(obuf, [dv, bv], val, mask=m)

        for t in range(nu):
            u = t * ns + sid
            f2 = lax.div(u, 8)
            d0 = cid * 32 + (u - f2 * 8) * 4
            pltpu.sync_copy(sh_bl.at[pl.ds(f2 * B, B)], bl_v)
            pltpu.sync_copy(sh_off.at[pl.ds(f2 * NBP, NBP)], off_v)

            start_chunk(jnp.int32(0), 0, sem0, sem2)

            @pl.loop(0, NCH, step=2)
            def _(k):
                wait_chunk(k, 0, sem0, sem2)
                start_chunk(k + 1, 1, sem1, sem3)
                process(k, 0)
                wait_chunk(k + 1, 1, sem1, sem3)

                @pl.when(k + 2 < NCH)
                def _():
                    start_chunk(k + 2, 0, sem0, sem2)

                process(k + 1, 1)

            pltpu.sync_copy(tail_hbm.at[f2, pl.ds(d0, 4), :], stage.at[0])
            process(NCH, 0)
            pltpu.sync_copy(obuf, out_hbm.at[f2, pl.ds(d0, 4), :])

    return sc_kernel


def kernel(dense, sparse, cat_embs, Wd, bd):
    info = plsc.get_sparse_core_info()
    dtok_t = _dense_proj_t(dense.T, Wd.T, bd)
    sc_k = _make_sc_kernel(info.num_cores, info.num_subcores)
    tab_t = jnp.transpose(cat_embs, (0, 2, 1))
    tail = jnp.pad(tab_t[:, :, NCH * W:], ((0, 0), (0, 0), (0, W - (CARD - NCH * W))))
    out3 = sc_k(
        sparse.T,
        tab_t,
        tail,
        dtok_t,
    )
    return jnp.transpose(out3, (2, 0, 1))
